# baseline (speedup 1.0000x reference)
import functools

import jax
import jax.numpy as jnp
import numpy as np
from jax import lax
from jax.experimental import pallas as pl
from jax.experimental.pallas import tpu as pltpu

try:
    from jax.experimental.pallas import tpu_sc as plsc
except ImportError:
    plsc = None

_INTERPRET = False

_SC_NC = 2
_SC_NS = 16
_SC_NW = _SC_NC * _SC_NS


def _pad_cols(a, mult=128):
    d = a.shape[-1]
    pd = -(-d // mult) * mult
    if pd == d:
        return a
    return jnp.pad(a, ((0, 0), (0, pd - d)))


def _sc_gather(table, idx):
    V, D = table.shape
    (Btot,) = idx.shape
    assert D % 128 == 0, D
    PB = -(-Btot // (8 * _SC_NW)) * (8 * _SC_NW)
    if PB != Btot:
        idx = jnp.pad(idx, (0, PB - Btot))
    b_per_w = PB // _SC_NW
    rows_cap = max(8, ((384 * 1024) // (D * 4)) // 8 * 8)
    C = min(b_per_w, rows_cap, 128)
    while b_per_w % C:
        C -= 8
    nchunk = b_per_w // C
    mesh = plsc.VectorSubcoreMesh(core_axis_name="c", subcore_axis_name="s")

    @functools.partial(
        pl.kernel,
        mesh=mesh,
        out_type=jax.ShapeDtypeStruct((PB, D), jnp.float32),
        scratch_types=[
            pltpu.VMEM((C,), jnp.int32),
            pltpu.VMEM((C, D), jnp.float32),
            pltpu.SemaphoreType.DMA,
        ],
    )
    def gk(table_hbm, idx_hbm, out_hbm, idx_v, rows_v, sem):
        wid = lax.axis_index("s") * _SC_NC + lax.axis_index("c")
        base = wid * b_per_w

        def body(c, carry):
            start = pl.multiple_of(base + c * C, 8)
            pltpu.sync_copy(idx_hbm.at[pl.ds(start, C)], idx_v)
            pltpu.async_copy(table_hbm.at[idx_v], rows_v, sem).wait()
            pltpu.sync_copy(rows_v, out_hbm.at[pl.ds(start, C)])
            return carry

        lax.fori_loop(0, nchunk, body, 0)

    out = gk(table, idx)
    return out[:Btot] if PB != Btot else out


def _dot(a, b):
    return jnp.dot(a, b, preferred_element_type=jnp.float32)


def _embed(x2, w1, b1, w2, b2):
    R, F = x2.shape
    H = w2.shape[1]

    def body(x_ref, w1r, b1r, w2r, b2r, o_ref):
        h = jax.nn.relu(_dot(x_ref[...], w1r[...]) + b1r[...])
        o_ref[...] = _dot(h, w2r[...]) + b2r[...]

    return pl.pallas_call(
        body,
        grid=(1,),
        in_specs=[
            pl.BlockSpec((R, F), lambda i: (0, 0)),
            pl.BlockSpec(w1.shape, lambda i: (0, 0)),
            pl.BlockSpec(b1.shape, lambda i: (0, 0)),
            pl.BlockSpec(w2.shape, lambda i: (0, 0)),
            pl.BlockSpec(b2.shape, lambda i: (0, 0)),
        ],
        out_specs=pl.BlockSpec((R, H), lambda i: (0, 0)),
        out_shape=jax.ShapeDtypeStruct((R, H), jnp.float32),
        interpret=_INTERPRET,
    )(x2, w1, b1, w2, b2)


def _proj(feat2, Wk, bk, Wv, bv):
    R, dp = feat2.shape
    D = Wk.shape[1]
    TM = min(512, R)

    def body(f_ref, wk, bkr, wv, bvr, kv_ref):
        f = f_ref[...]
        k = _dot(f, wk[...]) + bkr[...]
        v = _dot(f, wv[...]) + bvr[...]
        kv_ref[...] = jnp.concatenate([k, v], axis=1)

    full = lambda i: (0, 0)
    return pl.pallas_call(
        body,
        grid=(R // TM,),
        in_specs=[
            pl.BlockSpec((TM, dp), lambda i: (i, 0)),
            pl.BlockSpec(Wk.shape, full),
            pl.BlockSpec(bk.shape, full),
            pl.BlockSpec(Wv.shape, full),
            pl.BlockSpec(bv.shape, full),
        ],
        out_specs=pl.BlockSpec((TM, 2 * D), lambda i: (i, 0)),
        out_shape=jax.ShapeDtypeStruct((R, 2 * D), jnp.float32),
        interpret=_INTERPRET,
    )(feat2, Wk, bk, Wv, bv)


def _knn(xy_q, xy_cT):
    B, M, _ = xy_q.shape
    N = xy_cT.shape[2]
    K = min(16, N)
    TM = min(256, M)
    INF = np.float32(np.inf)

    def body(q_ref, c_ref, o_ref):
        b = pl.program_id(0)
        q = q_ref[0]
        qx = q[:, 0:1]
        qy = q[:, 1:2]
        cx = c_ref[0, 0:1, :]
        cy = c_ref[0, 1:2, :]
        d = (qx - cx) ** 2 + (qy - cy) ** 2
        iota = lax.broadcasted_iota(jnp.int32, (TM, N), 1)
        cols = []
        for _ in range(K):
            rowmin = jnp.min(d, axis=1, keepdims=True)
            cand = jnp.where(d == rowmin, iota, N)
            sel = jnp.min(cand, axis=1, keepdims=True)
            cols.append(sel)
            d = jnp.where(cand == sel, INF, d)
        o_ref[0] = jnp.concatenate(cols, axis=1) + b * N

    return pl.pallas_call(
        body,
        grid=(B, M // TM),
        in_specs=[
            pl.BlockSpec((1, TM, 2), lambda b, m: (b, m, 0)),
            pl.BlockSpec((1, 2, N), lambda b, m: (b, 0, 0)),
        ],
        out_specs=pl.BlockSpec((1, TM, K), lambda b, m: (b, m, 0)),
        out_shape=jax.ShapeDtypeStruct((B, M, K), jnp.int32),
        interpret=_INTERPRET,
    )(xy_q, xy_cT)


def _fps(X, Y, npoint):
    B, N = X.shape

    def body(x_ref, y_ref, o_ref):
        x = x_ref[...]
        y = y_ref[...]
        iota = lax.broadcasted_iota(jnp.int32, (B, N), 1)
        liota = lax.broadcasted_iota(jnp.int32, (B, npoint), 1)
        roff = lax.broadcasted_iota(jnp.int32, (B, 1), 0) * N

        def step(t, carry):
            distance, far, cents = carry
            cents = jnp.where(liota == t, far + roff, cents)
            m = iota == far
            cx = jnp.sum(jnp.where(m, x, 0.0), axis=1, keepdims=True)
            cy = jnp.sum(jnp.where(m, y, 0.0), axis=1, keepdims=True)
            dist = (x - cx) ** 2 + (y - cy) ** 2
            distance = jnp.minimum(distance, dist)
            dmax = jnp.max(distance, axis=1, keepdims=True)
            far = jnp.min(
                jnp.where(distance == dmax, iota, N), axis=1, keepdims=True
            )
            return distance, far, cents

        init = (
            jnp.full((B, N), 1e10, jnp.float32),
            jnp.zeros((B, 1), jnp.int32),
            jnp.zeros((B, npoint), jnp.int32),
        )
        _, _, cents = lax.fori_loop(0, npoint, step, init)
        o_ref[...] = cents

    return pl.pallas_call(
        body,
        grid=(1,),
        in_specs=[
            pl.BlockSpec((B, N), lambda i: (0, 0)),
            pl.BlockSpec((B, N), lambda i: (0, 0)),
        ],
        out_specs=pl.BlockSpec((B, npoint), lambda i: (0, 0)),
        out_shape=jax.ShapeDtypeStruct((B, npoint), jnp.int32),
        interpret=_INTERPRET,
    )(X, Y)


def _attn(g, xyp2, feat2, Wq, bq, d1p, d1b, d2, d2b, g1, g1b, g2s, g2bs, fc2, fc2b, K):
    R, dp = feat2.shape
    D = Wq.shape[1]
    GD = g.shape[1]
    TM = min(128, R)

    def body(g_ref, xy_ref, f_ref, wq, bqr, d1w, d1bb, d2w, d2bb,
             g1w, g1bb, g2w, g2bb, fw, fb, o_ref):
        gg = g_ref[...]
        kk = gg[:, :D]
        vv = gg[:, D:2 * D]
        gxy = gg[:, 2 * D:2 * D + 16]
        qxy = xy_ref[...]
        qxyr = jnp.broadcast_to(qxy[:, None, :], (TM, K, 16)).reshape(TM * K, 16)
        delta = qxyr - gxy
        pe = _dot(jax.nn.relu(_dot(delta, d1w[...]) + d1bb[...]), d2w[...]) + d2bb[...]
        q = _dot(f_ref[...], wq[...]) + bqr[...]
        qr = jnp.broadcast_to(q[:, None, :], (TM, K, D)).reshape(TM * K, D)
        z = qr - kk + pe
        a = _dot(jax.nn.relu(_dot(z, g1w[...]) + g1bb[...]), g2w[...]) + g2bb[...]
        a3 = a.reshape(TM, K, D)
        mx = jnp.max(a3, axis=1, keepdims=True)
        e = jnp.exp(a3 - mx)
        s = jnp.sum(e, axis=1, keepdims=True)
        attn = e / s
        vpe = (vv + pe).reshape(TM, K, D)
        res = jnp.sum(attn * vpe, axis=1)
        o_ref[...] = _dot(res, fw[...]) + fb[...] + f_ref[...]

    full = lambda i: (0, 0)
    return pl.pallas_call(
        body,
        grid=(R // TM,),
        in_specs=[
            pl.BlockSpec((TM * K, GD), lambda i: (i, 0)),
            pl.BlockSpec((TM, 16), lambda i: (i, 0)),
            pl.BlockSpec((TM, dp), lambda i: (i, 0)),
            pl.BlockSpec(Wq.shape, full),
            pl.BlockSpec(bq.shape, full),
            pl.BlockSpec(d1p.shape, full),
            pl.BlockSpec(d1b.shape, full),
            pl.BlockSpec(d2.shape, full),
            pl.BlockSpec(d2b.shape, full),
            pl.BlockSpec(g1.shape, full),
            pl.BlockSpec(g1b.shape, full),
            pl.BlockSpec(g2s.shape, full),
            pl.BlockSpec(g2bs.shape, full),
            pl.BlockSpec(fc2.shape, full),
            pl.BlockSpec(fc2b.shape, full),
        ],
        out_specs=pl.BlockSpec((TM, dp), lambda i: (i, 0)),
        out_shape=jax.ShapeDtypeStruct((R, dp), jnp.float32),
        interpret=_INTERPRET,
    )(g, xyp2, feat2, Wq, bq, d1p, d1b, d2, d2b, g1, g1b, g2s, g2bs, fc2, fc2b)


def _td_mlp(g, nxy, W1xy, W1p, b1, W2, b2, K):
    RK, GD = g.shape
    R = RK // K
    dp = W1p.shape[0]
    ch = W1p.shape[1]
    TM = min(128, R)

    def body(g_ref, n_ref, wxy, wp, b1r, w2r, b2r, o_ref):
        gg = g_ref[...]
        gp = gg[:, :dp]
        gxy = gg[:, dp:dp + 16]
        nrep = jnp.broadcast_to(n_ref[...][:, None, :], (TM, K, 16)).reshape(TM * K, 16)
        delta = gxy - nrep
        h = jax.nn.relu(_dot(delta, wxy[...]) + _dot(gp, wp[...]) + b1r[...])
        h = jax.nn.relu(_dot(h, w2r[...]) + b2r[...])
        o_ref[...] = jnp.max(h.reshape(TM, K, ch), axis=1)

    full = lambda i: (0, 0)
    return pl.pallas_call(
        body,
        grid=(R // TM,),
        in_specs=[
            pl.BlockSpec((TM * K, GD), lambda i: (i, 0)),
            pl.BlockSpec((TM, 16), lambda i: (i, 0)),
            pl.BlockSpec(W1xy.shape, full),
            pl.BlockSpec(W1p.shape, full),
            pl.BlockSpec(b1.shape, full),
            pl.BlockSpec(W2.shape, full),
            pl.BlockSpec(b2.shape, full),
        ],
        out_specs=pl.BlockSpec((TM, ch), lambda i: (i, 0)),
        out_shape=jax.ShapeDtypeStruct((R, ch), jnp.float32),
        interpret=_INTERPRET,
    )(g, nxy, W1xy, W1p, b1, W2, b2)


def _pad_xy(xy):
    B, N, _ = xy.shape
    return jnp.pad(xy, ((0, 0), (0, 0), (0, 14))).reshape(B * N, 16)


def _tblock(p, xyp2, gidx, feat):
    B, N, dp = feat.shape
    D = p["wq"].shape[0]
    K = min(16, N)
    Wq = p["fc1_w"] @ p["wq"]
    bq = (p["fc1_b"] @ p["wq"])[None]
    Wk = p["fc1_w"] @ p["wk"]
    bk = (p["fc1_b"] @ p["wk"])[None]
    Wv = p["fc1_w"] @ p["wv"]
    bv = (p["fc1_b"] @ p["wv"])[None]
    d1p = jnp.pad(p["d1_w"], ((0, 14), (0, 0)))
    scale = 1.0 / np.sqrt(D)
    feat2 = feat.reshape(B * N, dp)
    kv2 = _proj(feat2, Wk, bk, Wv, bv)
    table = _pad_cols(jnp.concatenate([kv2, xyp2], axis=1))
    g = _sc_gather(table, gidx.reshape(-1))
    out2 = _attn(
        g, xyp2, feat2, Wq, bq,
        d1p, p["d1_b"][None], p["d2_w"], p["d2_b"][None],
        p["g1_w"], p["g1_b"][None], p["g2_w"] * scale, p["g2_b"][None] * scale,
        p["fc2_w"], p["fc2_b"][None], K,
    )
    return out2.reshape(B, N, dp)


def _tdown(p, xyp2, gidx, nxy16, feat, M):
    B, N, dp = feat.shape
    K = min(16, N)
    eps = 1e-5
    s1 = p["bn1_g"] / np.sqrt(1.0 + eps)
    W1 = p["conv1_w"] * s1[None, :]
    b1 = (p["conv1_b"] * s1 + p["bn1_b"])[None]
    s2 = p["bn2_g"] / np.sqrt(1.0 + eps)
    W2 = p["conv2_w"] * s2[None, :]
    b2 = (p["conv2_b"] * s2 + p["bn2_b"])[None]
    W1xy = jnp.pad(W1[:2], ((0, 14), (0, 0)))
    W1p = W1[2:]
    table = _pad_cols(jnp.concatenate([feat.reshape(B * N, dp), xyp2], axis=1))
    g = _sc_gather(table, gidx.reshape(-1))
    h2 = _td_mlp(g, nxy16, W1xy, W1p, b1, W2, b2, K)
    ch = W1p.shape[1]
    return h2.reshape(B, M, ch)


def kernel(x, params):
    B, N, F = x.shape
    nstage = len(params["td"])
    xy = x[..., :2]

    xyp2s = [_pad_xy(xy)]
    gidx_tf = [_knn(xy, jnp.transpose(xy, (0, 2, 1)))]
    gidx_td = []
    nxy16s = []
    Ms = []
    cur_xy = xy
    for i in range(nstage):
        Ncur = cur_xy.shape[1]
        M = Ncur // 4
        Ms.append(M)
        fidx = _fps(cur_xy[..., 0], cur_xy[..., 1], M)
        nxy16 = _sc_gather(_pad_cols(xyp2s[-1]), fidx.reshape(-1))[:, :16]
        new_xy = nxy16[:, :2].reshape(B, M, 2)
        gidx_td.append(_knn(new_xy, jnp.transpose(cur_xy, (0, 2, 1))))
        gidx_tf.append(_knn(new_xy, jnp.transpose(new_xy, (0, 2, 1))))
        nxy16s.append(nxy16)
        xyp2s.append(_pad_xy(new_xy))
        cur_xy = new_xy

    h2 = _embed(
        x.reshape(B * N, F),
        params["fc1a_w"], params["fc1a_b"][None],
        params["fc1b_w"], params["fc1b_b"][None],
    )
    h = h2.reshape(B, N, -1)
    h = _tblock(params["t0"], xyp2s[0], gidx_tf[0], h)
    for i in range(nstage):
        h = _tdown(params["td"][i], xyp2s[i], gidx_td[i], nxy16s[i], h, Ms[i])
        h = _tblock(params["tf"][i], xyp2s[i + 1], gidx_tf[i + 1], h)
    return h


# speedup vs baseline: 1.0075x; 1.0075x over previous
import functools

import jax
import jax.numpy as jnp
import numpy as np
from jax import lax
from jax.experimental import pallas as pl
from jax.experimental.pallas import tpu as pltpu

try:
    from jax.experimental.pallas import tpu_sc as plsc
except ImportError:
    plsc = None

_INTERPRET = False

_SC_NC = 2
_SC_NS = 16
_SC_NW = _SC_NC * _SC_NS


def _pad_cols(a, mult=128):
    d = a.shape[-1]
    pd = -(-d // mult) * mult
    if pd == d:
        return a
    return jnp.pad(a, ((0, 0), (0, pd - d)))


def _sc_gather(table, idx):
    V, D = table.shape
    (Btot,) = idx.shape
    assert D % 128 == 0, D
    PB = -(-Btot // (8 * _SC_NW)) * (8 * _SC_NW)
    if PB != Btot:
        idx = jnp.pad(idx, (0, PB - Btot))
    b_per_w = PB // _SC_NW
    rows_cap = max(8, ((384 * 1024) // (D * 4)) // 8 * 8)
    C = min(b_per_w, rows_cap, 128)
    while b_per_w % C:
        C -= 8
    nchunk = b_per_w // C
    mesh = plsc.VectorSubcoreMesh(core_axis_name="c", subcore_axis_name="s")

    @functools.partial(
        pl.kernel,
        mesh=mesh,
        out_type=jax.ShapeDtypeStruct((PB, D), jnp.float32),
        scratch_types=[
            pltpu.VMEM((C,), jnp.int32),
            pltpu.VMEM((C, D), jnp.float32),
            pltpu.SemaphoreType.DMA,
        ],
    )
    def gk(table_hbm, idx_hbm, out_hbm, idx_v, rows_v, sem):
        wid = lax.axis_index("s") * _SC_NC + lax.axis_index("c")
        base = wid * b_per_w

        def body(c, carry):
            start = pl.multiple_of(base + c * C, 8)
            pltpu.sync_copy(idx_hbm.at[pl.ds(start, C)], idx_v)
            pltpu.async_copy(table_hbm.at[idx_v], rows_v, sem).wait()
            pltpu.sync_copy(rows_v, out_hbm.at[pl.ds(start, C)])
            return carry

        lax.fori_loop(0, nchunk, body, 0)

    out = gk(table, idx)
    return out[:Btot] if PB != Btot else out


def _dot(a, b):
    return jnp.dot(a, b, preferred_element_type=jnp.float32)


def _embed(x2, w1, b1, w2, b2):
    R, F = x2.shape
    H = w2.shape[1]

    def body(x_ref, w1r, b1r, w2r, b2r, o_ref):
        h = jax.nn.relu(_dot(x_ref[...], w1r[...]) + b1r[...])
        o_ref[...] = _dot(h, w2r[...]) + b2r[...]

    return pl.pallas_call(
        body,
        grid=(1,),
        in_specs=[
            pl.BlockSpec((R, F), lambda i: (0, 0)),
            pl.BlockSpec(w1.shape, lambda i: (0, 0)),
            pl.BlockSpec(b1.shape, lambda i: (0, 0)),
            pl.BlockSpec(w2.shape, lambda i: (0, 0)),
            pl.BlockSpec(b2.shape, lambda i: (0, 0)),
        ],
        out_specs=pl.BlockSpec((R, H), lambda i: (0, 0)),
        out_shape=jax.ShapeDtypeStruct((R, H), jnp.float32),
        interpret=_INTERPRET,
    )(x2, w1, b1, w2, b2)


def _proj(feat2, Wk, bk, Wv, bv):
    R, dp = feat2.shape
    D = Wk.shape[1]
    TM = min(512, R)

    def body(f_ref, wk, bkr, wv, bvr, kv_ref):
        f = f_ref[...]
        k = _dot(f, wk[...]) + bkr[...]
        v = _dot(f, wv[...]) + bvr[...]
        kv_ref[...] = jnp.concatenate([k, v], axis=1)

    full = lambda i: (0, 0)
    return pl.pallas_call(
        body,
        grid=(R // TM,),
        in_specs=[
            pl.BlockSpec((TM, dp), lambda i: (i, 0)),
            pl.BlockSpec(Wk.shape, full),
            pl.BlockSpec(bk.shape, full),
            pl.BlockSpec(Wv.shape, full),
            pl.BlockSpec(bv.shape, full),
        ],
        out_specs=pl.BlockSpec((TM, 2 * D), lambda i: (i, 0)),
        out_shape=jax.ShapeDtypeStruct((R, 2 * D), jnp.float32),
        interpret=_INTERPRET,
    )(feat2, Wk, bk, Wv, bv)


def _knn(xy_q, xy_cT):
    B, M, _ = xy_q.shape
    N = xy_cT.shape[2]
    K = min(16, N)
    TM = min(256, M)
    INF = np.float32(np.inf)

    def body(q_ref, c_ref, o_ref):
        b = pl.program_id(0)
        q = q_ref[0]
        qx = q[:, 0:1]
        qy = q[:, 1:2]
        cx = c_ref[0, 0:1, :]
        cy = c_ref[0, 1:2, :]
        d = (qx - cx) ** 2 + (qy - cy) ** 2
        iota = lax.broadcasted_iota(jnp.int32, (TM, N), 1)
        cols = []
        for _ in range(K):
            rowmin = jnp.min(d, axis=1, keepdims=True)
            cand = jnp.where(d == rowmin, iota, N)
            sel = jnp.min(cand, axis=1, keepdims=True)
            cols.append(sel)
            d = jnp.where(cand == sel, INF, d)
        o_ref[0] = jnp.concatenate(cols, axis=1) + b * N

    return pl.pallas_call(
        body,
        grid=(B, M // TM),
        in_specs=[
            pl.BlockSpec((1, TM, 2), lambda b, m: (b, m, 0)),
            pl.BlockSpec((1, 2, N), lambda b, m: (b, 0, 0)),
        ],
        out_specs=pl.BlockSpec((1, TM, K), lambda b, m: (b, m, 0)),
        out_shape=jax.ShapeDtypeStruct((B, M, K), jnp.int32),
        interpret=_INTERPRET,
    )(xy_q, xy_cT)


def _fps(X, Y, npoint):
    B, N = X.shape
    S = 8 if N % (8 * 8) == 0 else 1
    L = N // S

    def body(x_ref, y_ref, o_ref):
        x = x_ref[...].reshape(B, S, L)
        y = y_ref[...].reshape(B, S, L)
        io_s = lax.broadcasted_iota(jnp.int32, (B, S, L), 1)
        io_l = lax.broadcasted_iota(jnp.int32, (B, S, L), 2)
        iota = io_s * L + io_l
        liota = lax.broadcasted_iota(jnp.int32, (B, npoint), 1)
        roff = lax.broadcasted_iota(jnp.int32, (B, 1), 0) * N

        def _red(a, fn):
            return fn(fn(a, axis=2, keepdims=True), axis=1, keepdims=True)

        def step(t, carry):
            distance, far, cents = carry
            cents = jnp.where(liota == t, far + roff, cents)
            m = iota == far[:, :, None]
            cx = _red(jnp.where(m, x, 0.0), jnp.sum)
            cy = _red(jnp.where(m, y, 0.0), jnp.sum)
            dist = (x - cx) ** 2 + (y - cy) ** 2
            distance = jnp.minimum(distance, dist)
            dmax = _red(distance, jnp.max)
            far = _red(jnp.where(distance == dmax, iota, N), jnp.min)
            return distance, far.reshape(B, 1), cents

        init = (
            jnp.full((B, S, L), 1e10, jnp.float32),
            jnp.zeros((B, 1), jnp.int32),
            jnp.zeros((B, npoint), jnp.int32),
        )
        _, _, cents = lax.fori_loop(0, npoint, step, init)
        o_ref[...] = cents

    return pl.pallas_call(
        body,
        grid=(1,),
        in_specs=[
            pl.BlockSpec((B, N), lambda i: (0, 0)),
            pl.BlockSpec((B, N), lambda i: (0, 0)),
        ],
        out_specs=pl.BlockSpec((B, npoint), lambda i: (0, 0)),
        out_shape=jax.ShapeDtypeStruct((B, npoint), jnp.int32),
        interpret=_INTERPRET,
    )(X, Y)


def _attn(g, xyp2, feat2, Wq, bq, d1p, d1b, d2, d2b, g1, g1b, g2s, g2bs, fc2, fc2b, K):
    R, dp = feat2.shape
    D = Wq.shape[1]
    GD = g.shape[1]
    TM = min(128, R)

    def body(g_ref, xy_ref, f_ref, wq, bqr, d1w, d1bb, d2w, d2bb,
             g1w, g1bb, g2w, g2bb, fw, fb, o_ref):
        gg = g_ref[...]
        kk = gg[:, :D]
        vv = gg[:, D:2 * D]
        gxy = gg[:, 2 * D:2 * D + 16]
        qxy = xy_ref[...]
        qxyr = jnp.broadcast_to(qxy[:, None, :], (TM, K, 16)).reshape(TM * K, 16)
        delta = qxyr - gxy
        pe = _dot(jax.nn.relu(_dot(delta, d1w[...]) + d1bb[...]), d2w[...]) + d2bb[...]
        q = _dot(f_ref[...], wq[...]) + bqr[...]
        qr = jnp.broadcast_to(q[:, None, :], (TM, K, D)).reshape(TM * K, D)
        z = qr - kk + pe
        a = _dot(jax.nn.relu(_dot(z, g1w[...]) + g1bb[...]), g2w[...]) + g2bb[...]
        e = jnp.exp(a).reshape(TM, K, D)
        s = jnp.sum(e, axis=1)
        vpe = (vv + pe).reshape(TM, K, D)
        res = jnp.sum(e * vpe, axis=1) / s
        o_ref[...] = _dot(res, fw[...]) + fb[...] + f_ref[...]

    full = lambda i: (0, 0)
    return pl.pallas_call(
        body,
        grid=(R // TM,),
        in_specs=[
            pl.BlockSpec((TM * K, GD), lambda i: (i, 0)),
            pl.BlockSpec((TM, 16), lambda i: (i, 0)),
            pl.BlockSpec((TM, dp), lambda i: (i, 0)),
            pl.BlockSpec(Wq.shape, full),
            pl.BlockSpec(bq.shape, full),
            pl.BlockSpec(d1p.shape, full),
            pl.BlockSpec(d1b.shape, full),
            pl.BlockSpec(d2.shape, full),
            pl.BlockSpec(d2b.shape, full),
            pl.BlockSpec(g1.shape, full),
            pl.BlockSpec(g1b.shape, full),
            pl.BlockSpec(g2s.shape, full),
            pl.BlockSpec(g2bs.shape, full),
            pl.BlockSpec(fc2.shape, full),
            pl.BlockSpec(fc2b.shape, full),
        ],
        out_specs=pl.BlockSpec((TM, dp), lambda i: (i, 0)),
        out_shape=jax.ShapeDtypeStruct((R, dp), jnp.float32),
        interpret=_INTERPRET,
    )(g, xyp2, feat2, Wq, bq, d1p, d1b, d2, d2b, g1, g1b, g2s, g2bs, fc2, fc2b)


def _td_mlp(g, nxy, W1xy, W1p, b1, W2, b2, K):
    RK, GD = g.shape
    R = RK // K
    dp = W1p.shape[0]
    ch = W1p.shape[1]
    TM = min(128, R)

    def body(g_ref, n_ref, wxy, wp, b1r, w2r, b2r, o_ref):
        gg = g_ref[...]
        gp = gg[:, :dp]
        gxy = gg[:, dp:dp + 16]
        nrep = jnp.broadcast_to(n_ref[...][:, None, :], (TM, K, 16)).reshape(TM * K, 16)
        delta = gxy - nrep
        h = jax.nn.relu(_dot(delta, wxy[...]) + _dot(gp, wp[...]) + b1r[...])
        h = jax.nn.relu(_dot(h, w2r[...]) + b2r[...])
        o_ref[...] = jnp.max(h.reshape(TM, K, ch), axis=1)

    full = lambda i: (0, 0)
    return pl.pallas_call(
        body,
        grid=(R // TM,),
        in_specs=[
            pl.BlockSpec((TM * K, GD), lambda i: (i, 0)),
            pl.BlockSpec((TM, 16), lambda i: (i, 0)),
            pl.BlockSpec(W1xy.shape, full),
            pl.BlockSpec(W1p.shape, full),
            pl.BlockSpec(b1.shape, full),
            pl.BlockSpec(W2.shape, full),
            pl.BlockSpec(b2.shape, full),
        ],
        out_specs=pl.BlockSpec((TM, ch), lambda i: (i, 0)),
        out_shape=jax.ShapeDtypeStruct((R, ch), jnp.float32),
        interpret=_INTERPRET,
    )(g, nxy, W1xy, W1p, b1, W2, b2)


def _pad_xy(xy):
    B, N, _ = xy.shape
    return jnp.pad(xy, ((0, 0), (0, 0), (0, 14))).reshape(B * N, 16)


def _tblock(p, xyp2, gidx, feat):
    B, N, dp = feat.shape
    D = p["wq"].shape[0]
    K = min(16, N)
    Wq = p["fc1_w"] @ p["wq"]
    bq = (p["fc1_b"] @ p["wq"])[None]
    Wk = p["fc1_w"] @ p["wk"]
    bk = (p["fc1_b"] @ p["wk"])[None]
    Wv = p["fc1_w"] @ p["wv"]
    bv = (p["fc1_b"] @ p["wv"])[None]
    d1p = jnp.pad(p["d1_w"], ((0, 14), (0, 0)))
    scale = 1.0 / np.sqrt(D)
    feat2 = feat.reshape(B * N, dp)
    kv2 = _proj(feat2, Wk, bk, Wv, bv)
    table = _pad_cols(jnp.concatenate([kv2, xyp2], axis=1))
    g = _sc_gather(table, gidx.reshape(-1))
    out2 = _attn(
        g, xyp2, feat2, Wq, bq,
        d1p, p["d1_b"][None], p["d2_w"], p["d2_b"][None],
        p["g1_w"], p["g1_b"][None], p["g2_w"] * scale, p["g2_b"][None] * scale,
        p["fc2_w"], p["fc2_b"][None], K,
    )
    return out2.reshape(B, N, dp)


def _tdown(p, xyp2, gidx, nxy16, feat, M):
    B, N, dp = feat.shape
    K = min(16, N)
    eps = 1e-5
    s1 = p["bn1_g"] / np.sqrt(1.0 + eps)
    W1 = p["conv1_w"] * s1[None, :]
    b1 = (p["conv1_b"] * s1 + p["bn1_b"])[None]
    s2 = p["bn2_g"] / np.sqrt(1.0 + eps)
    W2 = p["conv2_w"] * s2[None, :]
    b2 = (p["conv2_b"] * s2 + p["bn2_b"])[None]
    W1xy = jnp.pad(W1[:2], ((0, 14), (0, 0)))
    W1p = W1[2:]
    table = _pad_cols(jnp.concatenate([feat.reshape(B * N, dp), xyp2], axis=1))
    g = _sc_gather(table, gidx.reshape(-1))
    h2 = _td_mlp(g, nxy16, W1xy, W1p, b1, W2, b2, K)
    ch = W1p.shape[1]
    return h2.reshape(B, M, ch)


def kernel(x, params):
    B, N, F = x.shape
    nstage = len(params["td"])
    xy = x[..., :2]

    xyp2s = [_pad_xy(xy)]
    gidx_tf = [_knn(xy, jnp.transpose(xy, (0, 2, 1)))]
    gidx_td = []
    nxy16s = []
    Ms = []
    cur_xy = xy
    for i in range(nstage):
        Ncur = cur_xy.shape[1]
        M = Ncur // 4
        Ms.append(M)
        fidx = _fps(cur_xy[..., 0], cur_xy[..., 1], M)
        nxy16 = _sc_gather(_pad_cols(xyp2s[-1]), fidx.reshape(-1))[:, :16]
        new_xy = nxy16[:, :2].reshape(B, M, 2)
        gidx_td.append(_knn(new_xy, jnp.transpose(cur_xy, (0, 2, 1))))
        gidx_tf.append(_knn(new_xy, jnp.transpose(new_xy, (0, 2, 1))))
        nxy16s.append(nxy16)
        xyp2s.append(_pad_xy(new_xy))
        cur_xy = new_xy

    h2 = _embed(
        x.reshape(B * N, F),
        params["fc1a_w"], params["fc1a_b"][None],
        params["fc1b_w"], params["fc1b_b"][None],
    )
    h = h2.reshape(B, N, -1)
    h = _tblock(params["t0"], xyp2s[0], gidx_tf[0], h)
    for i in range(nstage):
        h = _tdown(params["td"][i], xyp2s[i], gidx_td[i], nxy16s[i], h, Ms[i])
        h = _tblock(params["tf"][i], xyp2s[i + 1], gidx_tf[i + 1], h)
    return h


# speedup vs baseline: 1.0590x; 1.0512x over previous
import functools

import jax
import jax.numpy as jnp
import numpy as np
from jax import lax
from jax.experimental import pallas as pl
from jax.experimental.pallas import tpu as pltpu

try:
    from jax.experimental.pallas import tpu_sc as plsc
except ImportError:
    plsc = None

_INTERPRET = False

_SC_NC = 2
_SC_NS = 16
_SC_NW = _SC_NC * _SC_NS


def _pad_cols(a, mult=128):
    d = a.shape[-1]
    pd = -(-d // mult) * mult
    if pd == d:
        return a
    return jnp.pad(a, ((0, 0), (0, pd - d)))


def _sc_gather(table, idx):
    V, D = table.shape
    (Btot,) = idx.shape
    assert D % 128 == 0, D
    PB = -(-Btot // (8 * _SC_NW)) * (8 * _SC_NW)
    if PB != Btot:
        idx = jnp.pad(idx, (0, PB - Btot))
    b_per_w = PB // _SC_NW
    rows_cap = max(8, ((384 * 1024) // (D * 4)) // 8 * 8)
    C = min(b_per_w, rows_cap, 128)
    while b_per_w % C:
        C -= 8
    nchunk = b_per_w // C
    mesh = plsc.VectorSubcoreMesh(core_axis_name="c", subcore_axis_name="s")

    @functools.partial(
        pl.kernel,
        mesh=mesh,
        out_type=jax.ShapeDtypeStruct((PB, D), jnp.float32),
        scratch_types=[
            pltpu.VMEM((C,), jnp.int32),
            pltpu.VMEM((C, D), jnp.float32),
            pltpu.SemaphoreType.DMA,
        ],
    )
    def gk(table_hbm, idx_hbm, out_hbm, idx_v, rows_v, sem):
        wid = lax.axis_index("s") * _SC_NC + lax.axis_index("c")
        base = wid * b_per_w

        def body(c, carry):
            start = pl.multiple_of(base + c * C, 8)
            pltpu.sync_copy(idx_hbm.at[pl.ds(start, C)], idx_v)
            pltpu.async_copy(table_hbm.at[idx_v], rows_v, sem).wait()
            pltpu.sync_copy(rows_v, out_hbm.at[pl.ds(start, C)])
            return carry

        lax.fori_loop(0, nchunk, body, 0)

    out = gk(table, idx)
    return out[:Btot] if PB != Btot else out


def _dot(a, b):
    return jnp.dot(a, b, preferred_element_type=jnp.float32)


def _embed(x2, w1, b1, w2, b2):
    R, F = x2.shape
    H = w2.shape[1]

    def body(x_ref, w1r, b1r, w2r, b2r, o_ref):
        h = jax.nn.relu(_dot(x_ref[...], w1r[...]) + b1r[...])
        o_ref[...] = _dot(h, w2r[...]) + b2r[...]

    return pl.pallas_call(
        body,
        grid=(1,),
        in_specs=[
            pl.BlockSpec((R, F), lambda i: (0, 0)),
            pl.BlockSpec(w1.shape, lambda i: (0, 0)),
            pl.BlockSpec(b1.shape, lambda i: (0, 0)),
            pl.BlockSpec(w2.shape, lambda i: (0, 0)),
            pl.BlockSpec(b2.shape, lambda i: (0, 0)),
        ],
        out_specs=pl.BlockSpec((R, H), lambda i: (0, 0)),
        out_shape=jax.ShapeDtypeStruct((R, H), jnp.float32),
        interpret=_INTERPRET,
    )(x2, w1, b1, w2, b2)


def _proj(feat2, Wk, bk, Wv, bv):
    R, dp = feat2.shape
    D = Wk.shape[1]
    TM = min(512, R)

    def body(f_ref, wk, bkr, wv, bvr, kv_ref):
        f = f_ref[...]
        k = _dot(f, wk[...]) + bkr[...]
        v = _dot(f, wv[...]) + bvr[...]
        kv_ref[...] = jnp.concatenate([k, v], axis=1)

    full = lambda i: (0, 0)
    return pl.pallas_call(
        body,
        grid=(R // TM,),
        in_specs=[
            pl.BlockSpec((TM, dp), lambda i: (i, 0)),
            pl.BlockSpec(Wk.shape, full),
            pl.BlockSpec(bk.shape, full),
            pl.BlockSpec(Wv.shape, full),
            pl.BlockSpec(bv.shape, full),
        ],
        out_specs=pl.BlockSpec((TM, 2 * D), lambda i: (i, 0)),
        out_shape=jax.ShapeDtypeStruct((R, 2 * D), jnp.float32),
        interpret=_INTERPRET,
    )(feat2, Wk, bk, Wv, bv)


def _knn(xy_q, xy_cT):
    B, M, _ = xy_q.shape
    N = xy_cT.shape[2]
    K = min(16, N)
    TM = min(256, M)
    INF = np.float32(np.inf)

    def body(q_ref, c_ref, o_ref):
        b = pl.program_id(0)
        q = q_ref[0]
        qx = q[:, 0:1]
        qy = q[:, 1:2]
        cx = c_ref[0, 0:1, :]
        cy = c_ref[0, 1:2, :]
        d = (qx - cx) ** 2 + (qy - cy) ** 2
        iota = lax.broadcasted_iota(jnp.int32, (TM, N), 1)
        cols = []
        for _ in range(K):
            rowmin = jnp.min(d, axis=1, keepdims=True)
            cand = jnp.where(d == rowmin, iota, N)
            sel = jnp.min(cand, axis=1, keepdims=True)
            cols.append(sel)
            d = jnp.where(cand == sel, INF, d)
        o_ref[0] = jnp.concatenate(cols, axis=1) + b * N

    return pl.pallas_call(
        body,
        grid=(B, M // TM),
        in_specs=[
            pl.BlockSpec((1, TM, 2), lambda b, m: (b, m, 0)),
            pl.BlockSpec((1, 2, N), lambda b, m: (b, 0, 0)),
        ],
        out_specs=pl.BlockSpec((1, TM, K), lambda b, m: (b, m, 0)),
        out_shape=jax.ShapeDtypeStruct((B, M, K), jnp.int32),
        interpret=_INTERPRET,
    )(xy_q, xy_cT)


def _fps(X, Y, npoint):
    B, N = X.shape
    L = 128 if N % 128 == 0 else N
    S = N // L

    def body(x_ref, y_ref, o_ref):
        x = x_ref[...].reshape(B, 1, S, L)
        y = y_ref[...].reshape(B, 1, S, L)
        xy = jnp.concatenate([x, y], axis=1)
        io_s = lax.broadcasted_iota(jnp.int32, (B, S, L), 1)
        io_l = lax.broadcasted_iota(jnp.int32, (B, S, L), 2)
        iota_f = (io_s * L + io_l).astype(jnp.float32)
        liota = lax.broadcasted_iota(jnp.int32, (B, npoint), 1)
        roff = lax.broadcasted_iota(jnp.int32, (B, 1), 0) * N
        NF = np.float32(N)

        def _red2(a, fn):
            return fn(fn(a, axis=-1, keepdims=True), axis=-2, keepdims=True)

        def step(t, carry):
            distance, far_f, cents = carry
            far_i = far_f.reshape(B, 1).astype(jnp.int32)
            cents = jnp.where(liota == t, far_i + roff, cents)
            m = iota_f == far_f
            cxy = _red2(jnp.where(m[:, None], xy, 0.0), jnp.sum)
            dist = (x[:, 0] - cxy[:, 0]) ** 2 + (y[:, 0] - cxy[:, 1]) ** 2
            distance = jnp.minimum(distance, dist)
            dmax = _red2(distance, jnp.max)
            far_f = _red2(jnp.where(distance == dmax, iota_f, NF), jnp.min)
            return distance, far_f, cents

        init = (
            jnp.full((B, S, L), 1e10, jnp.float32),
            jnp.zeros((B, 1, 1), jnp.float32),
            jnp.zeros((B, npoint), jnp.int32),
        )
        _, _, cents = lax.fori_loop(0, npoint, step, init)
        o_ref[...] = cents

    return pl.pallas_call(
        body,
        grid=(1,),
        in_specs=[
            pl.BlockSpec((B, N), lambda i: (0, 0)),
            pl.BlockSpec((B, N), lambda i: (0, 0)),
        ],
        out_specs=pl.BlockSpec((B, npoint), lambda i: (0, 0)),
        out_shape=jax.ShapeDtypeStruct((B, npoint), jnp.int32),
        interpret=_INTERPRET,
    )(X, Y)


def _attn(g, xyp2, feat2, Wq, bq, d1p, d1b, d2, d2b, g1, g1b, g2s, g2bs, fc2, fc2b, K):
    R, dp = feat2.shape
    D = Wq.shape[1]
    GD = g.shape[1]
    TM = min(128, R)

    def body(g_ref, xy_ref, f_ref, wq, bqr, d1w, d1bb, d2w, d2bb,
             g1w, g1bb, g2w, g2bb, fw, fb, o_ref):
        gg = g_ref[...]
        kk = gg[:, :D]
        vv = gg[:, D:2 * D]
        gxy = gg[:, 2 * D:2 * D + 16]
        qxy = xy_ref[...]
        qxyr = jnp.broadcast_to(qxy[:, None, :], (TM, K, 16)).reshape(TM * K, 16)
        delta = qxyr - gxy
        pe = _dot(jax.nn.relu(_dot(delta, d1w[...]) + d1bb[...]), d2w[...]) + d2bb[...]
        q = _dot(f_ref[...], wq[...]) + bqr[...]
        qr = jnp.broadcast_to(q[:, None, :], (TM, K, D)).reshape(TM * K, D)
        z = qr - kk + pe
        a = _dot(jax.nn.relu(_dot(z, g1w[...]) + g1bb[...]), g2w[...]) + g2bb[...]
        e = jnp.exp(a).reshape(TM, K, D)
        s = jnp.sum(e, axis=1)
        vpe = (vv + pe).reshape(TM, K, D)
        res = jnp.sum(e * vpe, axis=1) / s
        o_ref[...] = _dot(res, fw[...]) + fb[...] + f_ref[...]

    full = lambda i: (0, 0)
    return pl.pallas_call(
        body,
        grid=(R // TM,),
        in_specs=[
            pl.BlockSpec((TM * K, GD), lambda i: (i, 0)),
            pl.BlockSpec((TM, 16), lambda i: (i, 0)),
            pl.BlockSpec((TM, dp), lambda i: (i, 0)),
            pl.BlockSpec(Wq.shape, full),
            pl.BlockSpec(bq.shape, full),
            pl.BlockSpec(d1p.shape, full),
            pl.BlockSpec(d1b.shape, full),
            pl.BlockSpec(d2.shape, full),
            pl.BlockSpec(d2b.shape, full),
            pl.BlockSpec(g1.shape, full),
            pl.BlockSpec(g1b.shape, full),
            pl.BlockSpec(g2s.shape, full),
            pl.BlockSpec(g2bs.shape, full),
            pl.BlockSpec(fc2.shape, full),
            pl.BlockSpec(fc2b.shape, full),
        ],
        out_specs=pl.BlockSpec((TM, dp), lambda i: (i, 0)),
        out_shape=jax.ShapeDtypeStruct((R, dp), jnp.float32),
        interpret=_INTERPRET,
    )(g, xyp2, feat2, Wq, bq, d1p, d1b, d2, d2b, g1, g1b, g2s, g2bs, fc2, fc2b)


def _td_mlp(g, nxy, W1xy, W1p, b1, W2, b2, K):
    RK, GD = g.shape
    R = RK // K
    dp = W1p.shape[0]
    ch = W1p.shape[1]
    TM = min(128, R)

    def body(g_ref, n_ref, wxy, wp, b1r, w2r, b2r, o_ref):
        gg = g_ref[...]
        gp = gg[:, :dp]
        gxy = gg[:, dp:dp + 16]
        nrep = jnp.broadcast_to(n_ref[...][:, None, :], (TM, K, 16)).reshape(TM * K, 16)
        delta = gxy - nrep
        h = jax.nn.relu(_dot(delta, wxy[...]) + _dot(gp, wp[...]) + b1r[...])
        h = jax.nn.relu(_dot(h, w2r[...]) + b2r[...])
        o_ref[...] = jnp.max(h.reshape(TM, K, ch), axis=1)

    full = lambda i: (0, 0)
    return pl.pallas_call(
        body,
        grid=(R // TM,),
        in_specs=[
            pl.BlockSpec((TM * K, GD), lambda i: (i, 0)),
            pl.BlockSpec((TM, 16), lambda i: (i, 0)),
            pl.BlockSpec(W1xy.shape, full),
            pl.BlockSpec(W1p.shape, full),
            pl.BlockSpec(b1.shape, full),
            pl.BlockSpec(W2.shape, full),
            pl.BlockSpec(b2.shape, full),
        ],
        out_specs=pl.BlockSpec((TM, ch), lambda i: (i, 0)),
        out_shape=jax.ShapeDtypeStruct((R, ch), jnp.float32),
        interpret=_INTERPRET,
    )(g, nxy, W1xy, W1p, b1, W2, b2)


def _pad_xy(xy):
    B, N, _ = xy.shape
    return jnp.pad(xy, ((0, 0), (0, 0), (0, 14))).reshape(B * N, 16)


def _tblock(p, xyp2, gidx, feat):
    B, N, dp = feat.shape
    D = p["wq"].shape[0]
    K = min(16, N)
    Wq = p["fc1_w"] @ p["wq"]
    bq = (p["fc1_b"] @ p["wq"])[None]
    Wk = p["fc1_w"] @ p["wk"]
    bk = (p["fc1_b"] @ p["wk"])[None]
    Wv = p["fc1_w"] @ p["wv"]
    bv = (p["fc1_b"] @ p["wv"])[None]
    d1p = jnp.pad(p["d1_w"], ((0, 14), (0, 0)))
    scale = 1.0 / np.sqrt(D)
    feat2 = feat.reshape(B * N, dp)
    kv2 = _proj(feat2, Wk, bk, Wv, bv)
    table = _pad_cols(jnp.concatenate([kv2, xyp2], axis=1))
    g = _sc_gather(table, gidx.reshape(-1))
    out2 = _attn(
        g, xyp2, feat2, Wq, bq,
        d1p, p["d1_b"][None], p["d2_w"], p["d2_b"][None],
        p["g1_w"], p["g1_b"][None], p["g2_w"] * scale, p["g2_b"][None] * scale,
        p["fc2_w"], p["fc2_b"][None], K,
    )
    return out2.reshape(B, N, dp)


def _tdown(p, xyp2, gidx, nxy16, feat, M):
    B, N, dp = feat.shape
    K = min(16, N)
    eps = 1e-5
    s1 = p["bn1_g"] / np.sqrt(1.0 + eps)
    W1 = p["conv1_w"] * s1[None, :]
    b1 = (p["conv1_b"] * s1 + p["bn1_b"])[None]
    s2 = p["bn2_g"] / np.sqrt(1.0 + eps)
    W2 = p["conv2_w"] * s2[None, :]
    b2 = (p["conv2_b"] * s2 + p["bn2_b"])[None]
    W1xy = jnp.pad(W1[:2], ((0, 14), (0, 0)))
    W1p = W1[2:]
    table = _pad_cols(jnp.concatenate([feat.reshape(B * N, dp), xyp2], axis=1))
    g = _sc_gather(table, gidx.reshape(-1))
    h2 = _td_mlp(g, nxy16, W1xy, W1p, b1, W2, b2, K)
    ch = W1p.shape[1]
    return h2.reshape(B, M, ch)


def kernel(x, params):
    B, N, F = x.shape
    nstage = len(params["td"])
    xy = x[..., :2]

    xyp2s = [_pad_xy(xy)]
    gidx_tf = [_knn(xy, jnp.transpose(xy, (0, 2, 1)))]
    gidx_td = []
    nxy16s = []
    Ms = []
    cur_xy = xy
    for i in range(nstage):
        Ncur = cur_xy.shape[1]
        M = Ncur // 4
        Ms.append(M)
        fidx = _fps(cur_xy[..., 0], cur_xy[..., 1], M)
        nxy16 = _sc_gather(_pad_cols(xyp2s[-1]), fidx.reshape(-1))[:, :16]
        new_xy = nxy16[:, :2].reshape(B, M, 2)
        gidx_td.append(_knn(new_xy, jnp.transpose(cur_xy, (0, 2, 1))))
        gidx_tf.append(_knn(new_xy, jnp.transpose(new_xy, (0, 2, 1))))
        nxy16s.append(nxy16)
        xyp2s.append(_pad_xy(new_xy))
        cur_xy = new_xy

    h2 = _embed(
        x.reshape(B * N, F),
        params["fc1a_w"], params["fc1a_b"][None],
        params["fc1b_w"], params["fc1b_b"][None],
    )
    h = h2.reshape(B, N, -1)
    h = _tblock(params["t0"], xyp2s[0], gidx_tf[0], h)
    for i in range(nstage):
        h = _tdown(params["td"][i], xyp2s[i], gidx_td[i], nxy16s[i], h, Ms[i])
        h = _tblock(params["tf"][i], xyp2s[i + 1], gidx_tf[i + 1], h)
    return h
